# baseline (speedup 1.0000x reference)
import functools

import jax
import jax.numpy as jnp
from jax import lax
from jax.experimental import pallas as pl
from jax.experimental.pallas import tpu as pltpu
from jax.experimental.pallas import tpu_sc as plsc

B = 4096
L = 200
D = 32
N_TOTAL = B * L
V = 1000000

NC = 2
NS = 16
NW = NC * NS
PER_W = N_TOTAL // NW
CHUNK = 128
N_CHUNK = PER_W // CHUNK

ROWS_BLK = 8000


def _tc_transform(table, wt, bias):
    def body(x_ref, w_ref, b_ref, o_ref):
        acc = jnp.dot(x_ref[...], w_ref[...],
                      preferred_element_type=jnp.float32)
        o_ref[...] = jnp.tanh(acc + b_ref[...])

    return pl.pallas_call(
        body,
        grid=(V // ROWS_BLK,),
        in_specs=[
            pl.BlockSpec((ROWS_BLK, 32), lambda i: (i, 0)),
            pl.BlockSpec((32, 32), lambda i: (0, 0)),
            pl.BlockSpec((1, 32), lambda i: (0, 0)),
        ],
        out_specs=pl.BlockSpec((ROWS_BLK, 32), lambda i: (i, 0)),
        out_shape=jax.ShapeDtypeStruct((V, D), jnp.float32),
    )(table, wt, bias)


def _sc_gather(table, idx3):
    mesh = plsc.VectorSubcoreMesh(core_axis_name="c", subcore_axis_name="s")

    @functools.partial(
        pl.kernel,
        out_type=jax.ShapeDtypeStruct((N_TOTAL, D), jnp.float32),
        mesh=mesh,
        scratch_types=[
            pltpu.VMEM((N_CHUNK, CHUNK), jnp.int32),
            pltpu.VMEM((CHUNK, D), jnp.float32),
            pltpu.VMEM((CHUNK, D), jnp.float32),
            pltpu.SemaphoreType.DMA,
            pltpu.SemaphoreType.DMA,
        ],
        compiler_params=pltpu.CompilerParams(use_tc_tiling_on_sc=False),
    )
    def k(table_hbm, idx_hbm, out_hbm, idx_v, rows_a, rows_b, sem_a, sem_b):
        wid = lax.axis_index("s") * NC + lax.axis_index("c")
        base = wid * PER_W
        pltpu.sync_copy(idx_hbm.at[wid], idx_v)

        pltpu.async_copy(table_hbm.at[idx_v.at[0]], rows_a, sem_a)

        @pl.loop(0, N_CHUNK // 2)
        def _(p):
            j = p * 2
            pltpu.async_copy(table_hbm.at[idx_v.at[j + 1]], rows_b, sem_b)
            pltpu.make_async_copy(table_hbm.at[idx_v.at[j]], rows_a, sem_a).wait()
            pltpu.sync_copy(rows_a, out_hbm.at[pl.ds(base + j * CHUNK, CHUNK)])

            @pl.when(j + 2 < N_CHUNK)
            def _():
                pltpu.async_copy(table_hbm.at[idx_v.at[j + 2]], rows_a, sem_a)

            pltpu.make_async_copy(table_hbm.at[idx_v.at[j + 1]], rows_b, sem_b).wait()
            pltpu.sync_copy(rows_b, out_hbm.at[pl.ds(base + (j + 1) * CHUNK, CHUNK)])

    return k(table, idx3)


BF = 32


def _tc_finish(rows):
    def body(g_ref, o_ref):
        o_ref[...] = g_ref[...].reshape(BF, L, D)

    return pl.pallas_call(
        body,
        grid=(B // BF,),
        in_specs=[pl.BlockSpec((BF * L, D), lambda i: (i, 0))],
        out_specs=pl.BlockSpec((BF, L, D), lambda i: (i, 0, 0)),
        out_shape=jax.ShapeDtypeStruct((B, L, D), jnp.float32),
    )(rows)


def kernel(x, table, W, b):
    tp = _tc_transform(table, W.T, b.reshape(1, D))

    idx3 = x.astype(jnp.int32).reshape(NW, N_CHUNK, CHUNK)
    out = _sc_gather(tp, idx3)
    return _tc_finish(out)


# speedup vs baseline: 2.3083x; 2.3083x over previous
import functools

import jax
import jax.numpy as jnp
from jax import lax
from jax.experimental import pallas as pl
from jax.experimental.pallas import tpu as pltpu
from jax.experimental.pallas import tpu_sc as plsc

B = 4096
L = 200
D = 32
N_TOTAL = B * L
V = 1000000

GRP = 8192
BLKR = GRP // 4
NGRP = -(-V // GRP)
VP = NGRP * BLKR * 4

NC = 2
NS = 16
NW = NC * NS
PER_W = N_TOTAL // NW
CHUNK = 128
N_CHUNK = PER_W // CHUNK


def _tc_transform(table_t, wt, bias):
    def body(x_ref, w_ref, b_ref, o_ref):
        w = w_ref[...]
        bb = b_ref[...]
        x = x_ref[...]
        cols = []
        for k in range(4):
            acc = lax.dot_general(
                x[:, k * BLKR:(k + 1) * BLKR], w, (((0,), (0,)), ((), ())),
                preferred_element_type=jnp.float32)
            cols.append(jnp.tanh(acc + bb))
        o_ref[...] = jnp.concatenate(cols, axis=1)

    return pl.pallas_call(
        body,
        grid=(NGRP,),
        in_specs=[
            pl.BlockSpec((32, GRP), lambda i: (0, i)),
            pl.BlockSpec((32, 32), lambda i: (0, 0)),
            pl.BlockSpec((1, 32), lambda i: (0, 0)),
        ],
        out_specs=pl.BlockSpec((BLKR, 128), lambda i: (i, 0)),
        out_shape=jax.ShapeDtypeStruct((VP // 4, 128), jnp.float32),
    )(table_t, wt, bias)


def _sc_gather(table, idx3):
    mesh = plsc.VectorSubcoreMesh(core_axis_name="c", subcore_axis_name="s")

    @functools.partial(
        pl.kernel,
        out_type=jax.ShapeDtypeStruct((N_TOTAL, D), jnp.float32),
        mesh=mesh,
        scratch_types=[
            pltpu.VMEM((N_CHUNK, CHUNK), jnp.int32),
            pltpu.VMEM((CHUNK, D), jnp.float32),
            pltpu.VMEM((CHUNK, D), jnp.float32),
            pltpu.SemaphoreType.DMA,
            pltpu.SemaphoreType.DMA,
        ],
        compiler_params=pltpu.CompilerParams(use_tc_tiling_on_sc=False),
    )
    def k(table_hbm, idx_hbm, out_hbm, idx_v, rows_a, rows_b, sem_a, sem_b):
        wid = lax.axis_index("s") * NC + lax.axis_index("c")
        base = wid * PER_W
        pltpu.sync_copy(idx_hbm.at[wid], idx_v)

        pltpu.async_copy(table_hbm.at[idx_v.at[0]], rows_a, sem_a)

        @pl.loop(0, N_CHUNK // 2)
        def _(p):
            j = p * 2
            pltpu.async_copy(table_hbm.at[idx_v.at[j + 1]], rows_b, sem_b)
            pltpu.make_async_copy(table_hbm.at[idx_v.at[j]], rows_a, sem_a).wait()
            pltpu.sync_copy(rows_a, out_hbm.at[pl.ds(base + j * CHUNK, CHUNK)])

            @pl.when(j + 2 < N_CHUNK)
            def _():
                pltpu.async_copy(table_hbm.at[idx_v.at[j + 2]], rows_a, sem_a)

            pltpu.make_async_copy(table_hbm.at[idx_v.at[j + 1]], rows_b, sem_b).wait()
            pltpu.sync_copy(rows_b, out_hbm.at[pl.ds(base + (j + 1) * CHUNK, CHUNK)])

    return k(table, idx3)


def kernel(x, table, W, b):
    t128 = _tc_transform(table.T, W.T, b.reshape(1, D))

    xi = x.astype(jnp.int32)
    g = xi >> 13
    w = xi & (GRP - 1)
    k = w >> 11
    r = w & (BLKR - 1)
    xj = (((g << 11) + r) << 2) + k

    idx3 = xj.reshape(NW, N_CHUNK, CHUNK)
    out = _sc_gather(t128.reshape(VP, D), idx3)
    return out.reshape(B, L, D)
